# baseline (speedup 1.0000x reference)
import functools

import jax
import jax.numpy as jnp
from jax import lax
from jax.experimental import pallas as pl
from jax.experimental.pallas import tpu as pltpu
from jax.experimental.pallas import tpu_sc as plsc

N = 10000
E = 160000
F = 256
HALF = 5120
NC = 2
NS = 16
SW = 2048
NSTRIP = 5
B = 64
CAP = SW + 2 * B
E_PAD = NS * NSTRIP * SW
RPT = HALF // NS


def _matmul_body(w_ref, x_ref, o_ref):
    o_ref[...] = jnp.dot(w_ref[...], x_ref[...],
                         preferred_element_type=jnp.float32)


def _support(weights, input_feature):
    return pl.pallas_call(
        _matmul_body,
        grid=(25,),
        in_specs=[
            pl.BlockSpec((400, F), lambda i: (i, 0)),
            pl.BlockSpec((F, F), lambda i: (0, 0)),
        ],
        out_specs=pl.BlockSpec((400, F), lambda i: (i, 0)),
        out_shape=jax.ShapeDtypeStruct((N, F), jnp.float32),
    )(weights, input_feature)


def _splat_lane(v, lane):
    idx = jnp.full((16,), lane, dtype=jnp.int32)
    return lax.gather(
        v, idx[:, None],
        dimension_numbers=lax.GatherDimensionNumbers(
            offset_dims=(), collapsed_slice_dims=(0,), start_index_map=(0,)),
        slice_sizes=(1,),
        mode=lax.GatherScatterMode.PROMISE_IN_BOUNDS)


_MESH = plsc.VectorSubcoreMesh(core_axis_name="c", subcore_axis_name="s")


@functools.partial(
    pl.kernel,
    out_type=jax.ShapeDtypeStruct((NC * HALF, F), jnp.float32),
    mesh=_MESH,
    scratch_types=[
        pltpu.VMEM((SW,), jnp.int32),
        pltpu.VMEM((SW,), jnp.int32),
        pltpu.VMEM((SW,), jnp.float32),
        pltpu.VMEM((CAP,), jnp.int32),
        pltpu.VMEM((CAP,), jnp.int32),
        pltpu.VMEM((CAP,), jnp.float32),
        [pltpu.VMEM((B, F), jnp.float32) for _ in range(2)],
        pltpu.VMEM((2, B), jnp.int32),
        [pltpu.SemaphoreType.DMA for _ in range(2)],
        [pltpu.SemaphoreType.DMA for _ in range(2)],
        pltpu.VMEM_SHARED((HALF, F), jnp.float32),
    ],
    compiler_params=pltpu.CompilerParams(use_tc_tiling_on_sc=False),
)
def _spmm(sup_hbm, rows_hbm, cols_hbm, vals_hbm, zeros_hbm, out_hbm,
          rows_raw, cols_raw, vals_raw, rows_c, cols_c, vals_c,
          bufs, stage, gsems, ssems, acc):
    c = lax.axis_index("c")
    s = lax.axis_index("s")
    bound = c * HALF

    pltpu.sync_copy(zeros_hbm, acc.at[pl.ds(s * RPT, RPT)])
    plsc.subcore_barrier()

    def g_start(k, b):
        pltpu.async_copy(sup_hbm.at[cols_c.at[pl.ds(b * B, B)]],
                         bufs[k], gsems[k])

    def g_wait(k):
        pltpu.make_async_copy(sup_hbm.at[stage.at[k]], bufs[k],
                              gsems[k]).wait()

    def s_start(k):
        pltpu.async_copy(bufs[k], acc.at[stage.at[k]], ssems[k], add=True)

    def s_wait(k):
        pltpu.make_async_copy(bufs[k], acc.at[stage.at[k]], ssems[k]).wait()

    def set_stage(k, b):
        for t in range(B // 16):
            stage[k, pl.ds(t * 16, 16)] = rows_c[pl.ds(b * B + t * 16, 16)]

    def scale(k, b):
        gbuf = bufs[k]

        def group_body(g, carry):
            vv = vals_c[pl.ds(b * B + g * 16, 16)]

            def edge_body(e, carry2):
                sc = _splat_lane(vv, e)
                row = g * 16 + e
                for f in range(F // 16):
                    sl = pl.ds(f * 16, 16)
                    gbuf[row, sl] = gbuf[row, sl] * sc
                return carry2

            lax.fori_loop(0, 16, edge_body, 0)
            return carry

        lax.fori_loop(0, B // 16, group_body, 0)

    def strip_body(st, carry):
        pltpu.sync_copy(rows_hbm.at[s, st], rows_raw)
        pltpu.sync_copy(cols_hbm.at[s, st], cols_raw)
        pltpu.sync_copy(vals_hbm.at[s, st], vals_raw)

        def cbody(g, off):
            sl = pl.ds(g * 16, 16)
            rv = rows_raw[sl]
            m = jnp.logical_and(rv >= bound, rv < bound + HALF)
            rows_c[sl] = jnp.where(m, rv - bound, 0)
            cols_c[sl] = jnp.where(m, cols_raw[sl], 0)
            vals_c[sl] = jnp.where(m, vals_raw[sl], 0.0)
            return off

        count = lax.fori_loop(0, SW // 16, cbody, jnp.int32(0))
        count = SW

        zi = jnp.zeros((16,), jnp.int32)
        zf = jnp.zeros((16,), jnp.float32)
        for t in range(2 * B // 16):
            w = pl.ds(count + t * 16, 16)
            rows_c[w] = zi
            cols_c[w] = zi
            vals_c[w] = zf

        npairs = (count + 2 * B - 1) // (2 * B)

        @pl.when(npairs > 0)
        def _pipeline():
            g_start(0, 0)

            def pbody(p, carry2):
                b0 = 2 * p
                b1 = b0 + 1
                g_wait(0)

                @pl.when(p > 0)
                def _():
                    s_wait(1)

                g_start(1, b1)
                scale(0, b0)
                set_stage(0, b0)
                s_start(0)

                g_wait(1)
                s_wait(0)

                @pl.when(p + 1 < npairs)
                def _():
                    g_start(0, b1 + 1)

                scale(1, b1)
                set_stage(1, b1)
                s_start(1)
                return carry2

            lax.fori_loop(0, npairs, pbody, 0)
            s_wait(1)

        return carry

    lax.fori_loop(0, NSTRIP, strip_body, 0)
    plsc.subcore_barrier()

    base = c * HALF + s * RPT
    pltpu.sync_copy(acc.at[pl.ds(s * RPT, RPT)],
                    out_hbm.at[pl.ds(base, RPT)])


@jax.jit
def kernel(adj_rows, adj_cols, adj_vals, input_feature, weights):
    support = _support(weights, input_feature)

    pad = E_PAD - E
    cols = jnp.concatenate(
        [adj_cols.astype(jnp.int32), jnp.zeros((pad,), jnp.int32)])
    rows = jnp.concatenate(
        [adj_rows.astype(jnp.int32), jnp.zeros((pad,), jnp.int32)])
    vals = jnp.concatenate([adj_vals, jnp.zeros((pad,), jnp.float32)])
    rows_r = rows.reshape(NS, NSTRIP, SW)
    cols_r = cols.reshape(NS, NSTRIP, SW)
    vals_r = vals.reshape(NS, NSTRIP, SW)
    zeros = jnp.zeros((RPT, F), jnp.float32)

    out2 = _spmm(support, rows_r, cols_r, vals_r, zeros)
    return out2[:N]


# speedup vs baseline: 14.8076x; 14.8076x over previous
import functools

import jax
import jax.numpy as jnp
from jax import lax
from jax.experimental import pallas as pl
from jax.experimental.pallas import tpu as pltpu
from jax.experimental.pallas import tpu_sc as plsc

N = 10000
E = 160000
F = 256
FH = 128
NC = 2
NS = 16
SW = 2048
NSTRIP = 5
B = 64
NQS = SW // B // 2
E_PAD = NS * NSTRIP * SW
N_PAD = 10240
RPT = N_PAD // NS
GROUPS = B // 16
FV = FH // 16


def _matmul_body(w_ref, x_ref, o_ref):
    o_ref[...] = jnp.dot(w_ref[...], x_ref[...],
                         preferred_element_type=jnp.float32)


def _support_halves(weights, input_feature):
    return pl.pallas_call(
        _matmul_body,
        grid=(NC, 25),
        in_specs=[
            pl.BlockSpec((400, F), lambda c, i: (i, 0)),
            pl.BlockSpec((F, FH), lambda c, i: (0, c)),
        ],
        out_specs=pl.BlockSpec((400, FH), lambda c, i: (c * 25 + i, 0)),
        out_shape=jax.ShapeDtypeStruct((NC * N, FH), jnp.float32),
    )(weights, input_feature)


def _splat_lane(v, lane):
    idx = jnp.full((16,), lane, dtype=jnp.int32)
    return lax.gather(
        v, idx[:, None],
        dimension_numbers=lax.GatherDimensionNumbers(
            offset_dims=(), collapsed_slice_dims=(0,), start_index_map=(0,)),
        slice_sizes=(1,),
        mode=lax.GatherScatterMode.PROMISE_IN_BOUNDS)


_MESH = plsc.VectorSubcoreMesh(core_axis_name="c", subcore_axis_name="s")


@functools.partial(
    pl.kernel,
    out_type=jax.ShapeDtypeStruct((NC * N_PAD, FH), jnp.float32),
    mesh=_MESH,
    scratch_types=[
        pltpu.VMEM((SW,), jnp.int32),
        pltpu.VMEM((SW,), jnp.int32),
        pltpu.VMEM((SW,), jnp.float32),
        [pltpu.VMEM((B, FH), jnp.float32) for _ in range(2)],
        pltpu.VMEM((2, B), jnp.int32),
        [pltpu.SemaphoreType.DMA for _ in range(2)],
        [pltpu.SemaphoreType.DMA for _ in range(2)],
        pltpu.VMEM_SHARED((N_PAD, FH), jnp.float32),
    ],
)
def _spmm(sup_hbm, cols_hbm, rows_hbm, vals_hbm, zeros_hbm, out_hbm,
          cols_raw, rows_raw, vals_raw, bufs, stage, gsems, ssems, acc):
    c = lax.axis_index("c")
    s = lax.axis_index("s")

    pltpu.sync_copy(zeros_hbm, acc.at[pl.ds(s * RPT, RPT)])
    plsc.subcore_barrier()

    def g_start(k, b):
        pltpu.async_copy(sup_hbm.at[cols_raw.at[pl.ds(b * B, B)]],
                         bufs[k], gsems[k])

    def g_wait(k):
        pltpu.make_async_copy(sup_hbm.at[stage.at[k]], bufs[k],
                              gsems[k]).wait()

    def s_start(k):
        pltpu.async_copy(bufs[k], acc.at[stage.at[k]], ssems[k], add=True)

    def s_wait(k):
        pltpu.make_async_copy(bufs[k], acc.at[stage.at[k]], ssems[k]).wait()

    def set_stage(k, b):
        for t in range(B // 16):
            stage[k, pl.ds(t * 16, 16)] = rows_raw[pl.ds(b * B + t * 16, 16)]

    def scale(k, b):
        gbuf = bufs[k]

        def group_body(g, carry):
            vv = vals_raw[pl.ds(b * B + g * 16, 16)]
            for e in range(16):
                sc = _splat_lane(vv, e)
                row = g * 16 + e
                for f in range(FV):
                    sl = pl.ds(f * 16, 16)
                    gbuf[row, sl] = gbuf[row, sl] * sc
            return carry

        lax.fori_loop(0, GROUPS, group_body, 0)

    def strip_body(st, carry):
        pltpu.sync_copy(cols_hbm.at[c, s, st], cols_raw)
        pltpu.sync_copy(rows_hbm.at[s, st], rows_raw)
        pltpu.sync_copy(vals_hbm.at[s, st], vals_raw)

        g_start(0, 0)

        def pair_body(q, carry2):
            b0 = 2 * q
            b1 = b0 + 1
            g_wait(0)

            @pl.when(q > 0)
            def _():
                s_wait(1)

            g_start(1, b1)
            scale(0, b0)
            set_stage(0, b0)
            s_start(0)

            g_wait(1)
            s_wait(0)

            @pl.when(q < NQS - 1)
            def _():
                g_start(0, b1 + 1)

            scale(1, b1)
            set_stage(1, b1)
            s_start(1)
            return carry2

        lax.fori_loop(0, NQS, pair_body, 0)
        s_wait(1)
        return carry

    lax.fori_loop(0, NSTRIP, strip_body, 0)
    plsc.subcore_barrier()

    base = c * N_PAD + s * RPT
    pltpu.sync_copy(acc.at[pl.ds(s * RPT, RPT)],
                    out_hbm.at[pl.ds(base, RPT)])


@jax.jit
def kernel(adj_rows, adj_cols, adj_vals, input_feature, weights):
    support = _support_halves(weights, input_feature)

    pad = E_PAD - E
    cols = jnp.concatenate(
        [adj_cols.astype(jnp.int32), jnp.zeros((pad,), jnp.int32)])
    rows = jnp.concatenate(
        [adj_rows.astype(jnp.int32), jnp.zeros((pad,), jnp.int32)])
    vals = jnp.concatenate([adj_vals, jnp.zeros((pad,), jnp.float32)])
    cols_r = cols.reshape(NS, NSTRIP, SW)
    cols2 = jnp.stack([cols_r, cols_r + N])
    rows_r = rows.reshape(NS, NSTRIP, SW)
    vals_r = vals.reshape(NS, NSTRIP, SW)
    zeros = jnp.zeros((RPT, FH), jnp.float32)

    out2 = _spmm(support, cols2, rows_r, vals_r, zeros)
    halves = out2.reshape(NC, N_PAD, FH)[:, :N]
    return halves.transpose(1, 0, 2).reshape(N, F)


# speedup vs baseline: 15.9419x; 1.0766x over previous
import functools

import jax
import jax.numpy as jnp
from jax import lax
from jax.experimental import pallas as pl
from jax.experimental.pallas import tpu as pltpu
from jax.experimental.pallas import tpu_sc as plsc

N = 10000
E = 160000
F = 256
FH = 128
NC = 2
NS = 16
SW = 2048
NSTRIP = 5
B = 128
NQS = SW // B // 2
E_PAD = NS * NSTRIP * SW
N_PAD = 10240
RPT = N_PAD // NS
GROUPS = B // 16
FV = FH // 16


def _matmul_body(w_ref, x_ref, o_ref):
    o_ref[...] = jnp.dot(w_ref[...], x_ref[...],
                         preferred_element_type=jnp.float32)


def _support_halves(weights, input_feature):
    return pl.pallas_call(
        _matmul_body,
        grid=(NC, 25),
        in_specs=[
            pl.BlockSpec((400, F), lambda c, i: (i, 0)),
            pl.BlockSpec((F, FH), lambda c, i: (0, c)),
        ],
        out_specs=pl.BlockSpec((400, FH), lambda c, i: (c * 25 + i, 0)),
        out_shape=jax.ShapeDtypeStruct((NC * N, FH), jnp.float32),
    )(weights, input_feature)


def _splat_lane(v, lane):
    idx = jnp.full((16,), lane, dtype=jnp.int32)
    return lax.gather(
        v, idx[:, None],
        dimension_numbers=lax.GatherDimensionNumbers(
            offset_dims=(), collapsed_slice_dims=(0,), start_index_map=(0,)),
        slice_sizes=(1,),
        mode=lax.GatherScatterMode.PROMISE_IN_BOUNDS)


_MESH = plsc.VectorSubcoreMesh(core_axis_name="c", subcore_axis_name="s")


@functools.partial(
    pl.kernel,
    out_type=jax.ShapeDtypeStruct((NC * N_PAD, FH), jnp.float32),
    mesh=_MESH,
    scratch_types=[
        pltpu.VMEM((SW,), jnp.int32),
        pltpu.VMEM((SW,), jnp.int32),
        pltpu.VMEM((SW,), jnp.float32),
        [pltpu.VMEM((B, FH), jnp.float32) for _ in range(2)],
        pltpu.VMEM((2, B), jnp.int32),
        [pltpu.SemaphoreType.DMA for _ in range(2)],
        [pltpu.SemaphoreType.DMA for _ in range(2)],
        pltpu.VMEM_SHARED((N_PAD, FH), jnp.float32),
    ],
)
def _spmm(sup_hbm, cols_hbm, rows_hbm, vals_hbm, zeros_hbm, out_hbm,
          cols_raw, rows_raw, vals_raw, bufs, stage, gsems, ssems, acc):
    c = lax.axis_index("c")
    s = lax.axis_index("s")

    pltpu.sync_copy(zeros_hbm, acc.at[pl.ds(s * RPT, RPT)])
    plsc.subcore_barrier()

    def g_start(k, b):
        pltpu.async_copy(sup_hbm.at[cols_raw.at[pl.ds(b * B, B)]],
                         bufs[k], gsems[k])

    def g_wait(k):
        pltpu.make_async_copy(sup_hbm.at[stage.at[k]], bufs[k],
                              gsems[k]).wait()

    def s_start(k):
        pltpu.async_copy(bufs[k], acc.at[stage.at[k]], ssems[k], add=True)

    def s_wait(k):
        pltpu.make_async_copy(bufs[k], acc.at[stage.at[k]], ssems[k]).wait()

    def set_stage(k, b):
        for t in range(B // 16):
            stage[k, pl.ds(t * 16, 16)] = rows_raw[pl.ds(b * B + t * 16, 16)]

    def scale(k, b):
        gbuf = bufs[k]

        def group_body(g, carry):
            vv = vals_raw[pl.ds(b * B + g * 16, 16)]
            for e in range(16):
                sc = _splat_lane(vv, e)
                row = g * 16 + e
                for f in range(FV):
                    sl = pl.ds(f * 16, 16)
                    gbuf[row, sl] = gbuf[row, sl] * sc
            return carry

        lax.fori_loop(0, GROUPS, group_body, 0)

    def strip_body(st, carry):
        pltpu.sync_copy(cols_hbm.at[c, s, st], cols_raw)
        pltpu.sync_copy(rows_hbm.at[s, st], rows_raw)
        pltpu.sync_copy(vals_hbm.at[s, st], vals_raw)

        g_start(0, 0)

        def pair_body(q, carry2):
            b0 = 2 * q
            b1 = b0 + 1
            g_wait(0)

            @pl.when(q > 0)
            def _():
                s_wait(1)

            g_start(1, b1)
            scale(0, b0)
            set_stage(0, b0)
            s_start(0)

            g_wait(1)
            s_wait(0)

            @pl.when(q < NQS - 1)
            def _():
                g_start(0, b1 + 1)

            scale(1, b1)
            set_stage(1, b1)
            s_start(1)
            return carry2

        lax.fori_loop(0, NQS, pair_body, 0)
        s_wait(1)
        return carry

    lax.fori_loop(0, NSTRIP, strip_body, 0)
    plsc.subcore_barrier()

    base = c * N_PAD + s * RPT
    pltpu.sync_copy(acc.at[pl.ds(s * RPT, RPT)],
                    out_hbm.at[pl.ds(base, RPT)])


@jax.jit
def kernel(adj_rows, adj_cols, adj_vals, input_feature, weights):
    support = _support_halves(weights, input_feature)

    pad = E_PAD - E
    cols = jnp.concatenate(
        [adj_cols.astype(jnp.int32), jnp.zeros((pad,), jnp.int32)])
    rows = jnp.concatenate(
        [adj_rows.astype(jnp.int32), jnp.zeros((pad,), jnp.int32)])
    vals = jnp.concatenate([adj_vals, jnp.zeros((pad,), jnp.float32)])
    cols_r = cols.reshape(NS, NSTRIP, SW)
    cols2 = jnp.stack([cols_r, cols_r + N])
    rows_r = rows.reshape(NS, NSTRIP, SW)
    vals_r = vals.reshape(NS, NSTRIP, SW)
    zeros = jnp.zeros((RPT, FH), jnp.float32)

    out2 = _spmm(support, cols2, rows_r, vals_r, zeros)
    halves = out2.reshape(NC, N_PAD, FH)[:, :N]
    return halves.transpose(1, 0, 2).reshape(N, F)
